# baseline (speedup 1.0000x reference)
import functools

import jax
import jax.numpy as jnp
from jax import lax
from jax.experimental import pallas as pl
from jax.experimental.pallas import tpu as pltpu
from jax.experimental.pallas import tpu_sc as plsc

_WINDOW = 128


def _sc_gather(W, idx_flat):
    n = idx_flat.shape[0]
    E = W.shape[1]
    idx2 = idx_flat.reshape(1, n)
    mesh = plsc.VectorSubcoreMesh(core_axis_name="core",
                                  subcore_axis_name="subcore")

    @functools.partial(
        pl.kernel,
        out_type=jax.ShapeDtypeStruct((n, E), W.dtype),
        mesh=mesh,
        compiler_params=pltpu.CompilerParams(use_tc_tiling_on_sc=False),
    )
    def gather_kernel(w_hbm, i_hbm, o_hbm):
        def body(i_vmem, o_vmem):
            pltpu.sync_copy(w_hbm.at[i_vmem.at[0]], o_vmem)

        pltpu.emit_pipeline(
            body,
            grid=(n // _WINDOW,),
            in_specs=[pl.BlockSpec((1, _WINDOW), index_map=lambda i: (0, i))],
            out_specs=[pl.BlockSpec((_WINDOW, E), index_map=lambda i: (i, 0))],
            core_axis_name=("core", "subcore"),
            dimension_semantics=(pltpu.PARALLEL,),
        )(i_hbm, o_hbm)

    return gather_kernel(W, idx2)


_BB = 8


def _mask_kernel(iv_ref, out_ref):
    iv = iv_ref[...]
    pad = iv != 0
    bb, _, l, _ = out_ref.shape
    row = lax.broadcasted_iota(jnp.int32, (bb, 1, l, l), 2)
    col = lax.broadcasted_iota(jnp.int32, (bb, 1, l, l), 3)
    out_ref[...] = pad[:, None, None, :] & (col <= row)


def _tc_mask(input_var):
    B, L = input_var.shape
    return pl.pallas_call(
        _mask_kernel,
        grid=(B // _BB,),
        in_specs=[pl.BlockSpec((_BB, L), lambda i: (i, 0))],
        out_specs=pl.BlockSpec((_BB, 1, L, L), lambda i: (i, 0, 0, 0)),
        out_shape=jax.ShapeDtypeStruct((B, 1, L, L), jnp.bool_),
    )(input_var)


def kernel(input_var, W):
    B, L = input_var.shape
    E = W.shape[1]
    embedded = _sc_gather(W, input_var.reshape(B * L)).reshape(B, L, E)
    tgt_mask = _tc_mask(input_var)
    return (embedded, tgt_mask)
